# Initial kernel scaffold; baseline (speedup 1.0000x reference)
import jax
import jax.numpy as jnp
from jax.experimental import pallas as pl


def kernel(edge_index, node_type_s, node_type_r):
    raise NotImplementedError("write your pallas kernel here")


# speedup vs baseline: 2.8324x; 2.8324x over previous
import functools

import jax
import jax.numpy as jnp
from jax import lax
from jax.experimental import pallas as pl
from jax.experimental.pallas import tpu as pltpu
from jax.experimental.pallas import tpu_sc as plsc

D = 8
DP = 16
DD = D * D
NC = 2
NS = 16
NW = NC * NS
CHUNK = 400


def _make(n_edges):
    e_per_w = n_edges // NW
    n_chunks = e_per_w // CHUNK
    mesh = plsc.VectorSubcoreMesh(core_axis_name="c", subcore_axis_name="s")

    def body(si_hbm, ri_hbm, s_hbm, r_hbm, out_hbm,
             sidx_v, ridx_v, srow_v, rrow_v, out_v, sem_s, sem_r):
        wid = lax.axis_index("s") * NC + lax.axis_index("c")
        base_w = wid * e_per_w

        def chunk_body(k, carry):
            base = base_w + k * CHUNK
            pltpu.sync_copy(si_hbm.at[pl.ds(base, CHUNK)], sidx_v)
            pltpu.sync_copy(ri_hbm.at[pl.ds(base, CHUNK)], ridx_v)
            cs = pltpu.async_copy(s_hbm.at[sidx_v], srow_v, sem_s)
            cr = pltpu.async_copy(r_hbm.at[ridx_v], rrow_v, sem_r)
            cs.wait()
            cr.wait()

            def grp_body(g, c2):
                c_vec = g * 16 + lax.iota(jnp.int32, 16)
                svals = [
                    plsc.load_gather(srow_v, [c_vec, jnp.full((16,), i, jnp.int32)])
                    for i in range(D)
                ]
                rvals = [
                    plsc.load_gather(rrow_v, [c_vec, jnp.full((16,), j, jnp.int32)])
                    for j in range(D)
                ]
                for i in range(D):
                    for j in range(D):
                        plsc.store_scatter(
                            out_v,
                            [c_vec, jnp.full((16,), i * D + j, jnp.int32)],
                            svals[i] * rvals[j],
                        )
                return c2

            lax.fori_loop(0, CHUNK // 16, grp_body, 0, unroll=False)
            pltpu.sync_copy(out_v, out_hbm.at[pl.ds(base, CHUNK)])
            return carry

        lax.fori_loop(0, n_chunks, chunk_body, 0, unroll=False)

    return pl.kernel(
        body,
        out_type=jax.ShapeDtypeStruct((n_edges, DD), jnp.float32),
        mesh=mesh,
        compiler_params=pltpu.CompilerParams(
            needs_layout_passes=False, use_tc_tiling_on_sc=False
        ),
        scratch_types=[
            pltpu.VMEM((CHUNK,), jnp.int32),
            pltpu.VMEM((CHUNK,), jnp.int32),
            pltpu.VMEM((CHUNK, DP), jnp.float32),
            pltpu.VMEM((CHUNK, DP), jnp.float32),
            pltpu.VMEM((CHUNK, DD), jnp.float32),
            pltpu.SemaphoreType.DMA,
            pltpu.SemaphoreType.DMA,
        ],
    )


def kernel(edge_index, node_type_s, node_type_r=None):
    if node_type_r is None:
        node_type_r = node_type_s
    n_edges = edge_index.shape[1]
    pad = ((0, 0), (0, DP - D))
    s_p = jnp.pad(node_type_s, pad)
    r_p = jnp.pad(node_type_r, pad)
    f = _make(n_edges)
    return f(edge_index[0], edge_index[1], s_p, r_p)


# speedup vs baseline: 4.5591x; 1.6096x over previous
import functools

import jax
import jax.numpy as jnp
import numpy as np
from jax import lax
from jax.experimental import pallas as pl
from jax.experimental.pallas import tpu as pltpu
from jax.experimental.pallas import tpu_sc as plsc

D = 8
DP = 16
DD = D * D
NC = 2
NS = 16
NW = NC * NS
CHUNK = 1000

_GDN = lax.GatherDimensionNumbers(
    offset_dims=(), collapsed_slice_dims=(0,), start_index_map=(0,)
)


def _lane_shuffle(v, idx):
    return lax.gather(
        v, idx.reshape(16, 1), _GDN, slice_sizes=(1,),
        mode=lax.GatherScatterMode.PROMISE_IN_BOUNDS,
    )


def _make(n_edges):
    e_per_w = n_edges // NW
    n_chunks = e_per_w // CHUNK
    mesh = plsc.VectorSubcoreMesh(core_axis_name="c", subcore_axis_name="s")

    def body(si_hbm, ri_hbm, s_hbm, r_hbm, out_hbm,
             sidx_v, ridx_v, srow_v, rrow_v, out_v, sem_s, sem_r):
        wid = lax.axis_index("s") * NC + lax.axis_index("c")
        base_w = wid * e_per_w
        lane = lax.iota(jnp.int32, 16)
        r_rep_idx = lane % D
        s_pat_idx = [lane // D + 2 * k for k in range(4)]

        def chunk_body(k, carry):
            base = base_w + k * CHUNK
            pltpu.sync_copy(si_hbm.at[pl.ds(base, CHUNK)], sidx_v)
            pltpu.sync_copy(ri_hbm.at[pl.ds(base, CHUNK)], ridx_v)
            cs = pltpu.async_copy(s_hbm.at[sidx_v], srow_v, sem_s)
            cr = pltpu.async_copy(r_hbm.at[ridx_v], rrow_v, sem_r)
            cs.wait()
            cr.wait()

            def edge_body(c, c2):
                s_vec = srow_v[c]
                r_vec = rrow_v[c]
                r_rep = _lane_shuffle(r_vec, r_rep_idx)
                for k in range(4):
                    s_pat = _lane_shuffle(s_vec, s_pat_idx[k])
                    out_v[c, pl.ds(k * 16, 16)] = s_pat * r_rep
                return c2

            lax.fori_loop(0, CHUNK, edge_body, 0, unroll=4)
            pltpu.sync_copy(out_v, out_hbm.at[pl.ds(base, CHUNK)])
            return carry

        lax.fori_loop(0, n_chunks, chunk_body, 0, unroll=False)

    return pl.kernel(
        body,
        out_type=jax.ShapeDtypeStruct((n_edges, DD), jnp.float32),
        mesh=mesh,
        compiler_params=pltpu.CompilerParams(
            needs_layout_passes=False, use_tc_tiling_on_sc=False
        ),
        scratch_types=[
            pltpu.VMEM((CHUNK,), jnp.int32),
            pltpu.VMEM((CHUNK,), jnp.int32),
            pltpu.VMEM((CHUNK, DP), jnp.float32),
            pltpu.VMEM((CHUNK, DP), jnp.float32),
            pltpu.VMEM((CHUNK, DD), jnp.float32),
            pltpu.SemaphoreType.DMA,
            pltpu.SemaphoreType.DMA,
        ],
    )


def kernel(edge_index, node_type_s, node_type_r=None):
    if node_type_r is None:
        node_type_r = node_type_s
    n_edges = edge_index.shape[1]
    pad = ((0, 0), (0, DP - D))
    s_p = jnp.pad(node_type_s, pad)
    r_p = jnp.pad(node_type_r, pad)
    f = _make(n_edges)
    return f(edge_index[0], edge_index[1], s_p, r_p)
